# baseline (speedup 1.0000x reference)
import functools

import jax
import jax.numpy as jnp
from jax import lax
from jax.experimental import pallas as pl
from jax.experimental.pallas import tpu as pltpu
from jax.experimental.pallas import tpu_sc as plsc

WORD_CNT = 100000
DIM = 128
BATCH = 4096
HIST = 200
N = BATCH * HIST

_info = plsc.get_sparse_core_info()
NC, NS = _info.num_cores, _info.num_subcores
NW = NC * NS
PER_W = N // NW
CHUNK = 128
NBUF = 2
NCHUNK = PER_W // CHUNK
NGROUP = NCHUNK // NBUF


SUB = 2
NBIG = NCHUNK // SUB
NRING = NBIG // NBUF


@functools.partial(
    pl.kernel,
    mesh=plsc.VectorSubcoreMesh(core_axis_name="c", subcore_axis_name="s"),
    out_type=jax.ShapeDtypeStruct((NW, NBIG, SUB * CHUNK, DIM), jnp.float32),
    scratch_types=[
        pltpu.VMEM((PER_W,), jnp.int32),
        pltpu.VMEM((NBUF, SUB * CHUNK, DIM), jnp.float32),
    ]
    + [pltpu.SemaphoreType.DMA] * (2 * NBUF),
)
def _gather_kernel(idx_hbm, table_hbm, out_hbm, idx_v, rows_v, *sems):
    gsem = sems[:NBUF]
    osem = sems[NBUF:]
    wid = lax.axis_index("s") * NC + lax.axis_index("c")

    pltpu.sync_copy(idx_hbm.at[wid], idx_v)

    def group_body(p, carry):
        gathers = []
        for b in range(NBUF):
            g = p * NBUF + b

            @pl.when(p > 0)
            def _():
                pltpu.make_async_copy(
                    rows_v.at[b], out_hbm.at[wid, 0], osem[b]
                ).wait()

            gathers.append(
                pltpu.async_copy(
                    table_hbm.at[idx_v.at[pl.ds(g * (SUB * CHUNK), SUB * CHUNK)]], rows_v.at[b], gsem[b]
                )
            )
        for b in range(NBUF):
            g = p * NBUF + b
            gathers[b].wait()
            pltpu.async_copy(rows_v.at[b], out_hbm.at[wid, g], osem[b])
        return carry

    lax.fori_loop(0, NRING, group_body, 0)

    for b in range(NBUF):
        pltpu.make_async_copy(rows_v.at[b], out_hbm.at[wid, 0], osem[b]).wait()


def kernel(x, rel_emb_weight):
    idx = x.reshape(NW, PER_W)
    out = _gather_kernel(idx, rel_emb_weight)
    return out.reshape(BATCH, HIST, DIM)


# speedup vs baseline: 1.0262x; 1.0262x over previous
import functools

import jax
import jax.numpy as jnp
from jax import lax
from jax.experimental import pallas as pl
from jax.experimental.pallas import tpu as pltpu
from jax.experimental.pallas import tpu_sc as plsc

WORD_CNT = 100000
DIM = 128
BATCH = 4096
HIST = 200
N = BATCH * HIST

_info = plsc.get_sparse_core_info()
NC, NS = _info.num_cores, _info.num_subcores
NW = NC * NS
PER_W = N // NW
CHUNK = 128
NBUF = 4
NCHUNK = PER_W // CHUNK
NTURN = NCHUNK // NBUF


@functools.partial(
    pl.kernel,
    mesh=plsc.VectorSubcoreMesh(core_axis_name="c", subcore_axis_name="s"),
    out_type=jax.ShapeDtypeStruct((N, DIM), jnp.float32),
    scratch_types=[
        pltpu.VMEM((NCHUNK, CHUNK), jnp.int32),
        pltpu.VMEM((NBUF, CHUNK, DIM), jnp.float32),
    ]
    + [pltpu.SemaphoreType.DMA] * (2 * NBUF),
)
def _gather_kernel(idx_hbm, table_hbm, out_hbm, idx_v, rows_v, *sems):
    gsem = sems[:NBUF]
    osem = sems[NBUF:]
    wid = lax.axis_index("s") * NC + lax.axis_index("c")
    base = wid * PER_W

    pltpu.sync_copy(idx_hbm.at[wid], idx_v)

    for b in range(NBUF):
        pltpu.async_copy(table_hbm.at[idx_v.at[b]], rows_v.at[b], gsem[b])

    def turn_body(p, carry):
        for b in range(NBUF):
            g = p * NBUF + b
            pltpu.make_async_copy(
                table_hbm.at[idx_v.at[g]], rows_v.at[b], gsem[b]
            ).wait()
            pltpu.async_copy(
                rows_v.at[b], out_hbm.at[pl.ds(base + g * CHUNK, CHUNK)], osem[b]
            )
            pltpu.make_async_copy(
                rows_v.at[b], out_hbm.at[pl.ds(base, CHUNK)], osem[b]
            ).wait()

            @pl.when(g + NBUF < NCHUNK)
            def _():
                pltpu.async_copy(
                    table_hbm.at[idx_v.at[g + NBUF]], rows_v.at[b], gsem[b]
                )

        return carry

    lax.fori_loop(0, NTURN, turn_body, 0)


def kernel(x, rel_emb_weight):
    idx = x.reshape(NW, NCHUNK, CHUNK)
    out = _gather_kernel(idx, rel_emb_weight)
    return out.reshape(BATCH, HIST, DIM)
